# baseline (speedup 1.0000x reference)
import functools

import jax
import jax.numpy as jnp
from jax import lax
from jax.experimental import pallas as pl
from jax.experimental.pallas import tpu as pltpu
from jax.experimental.pallas import tpu_sc as plsc

NC = 2
NS = 16
L = 16
NW = NC * NS
K = 128
IG = 8
NB = 2


def _sc_spmm(h, colw, roww, valw):
    N, D = h.shape
    cpw = colw.shape[1]
    ngroups = cpw // IG
    mesh = plsc.VectorSubcoreMesh(core_axis_name="c", subcore_axis_name="s")
    rps = ((-(-N // NS)) + 7) // 8 * 8
    ring = 2 * IG

    @functools.partial(
        pl.kernel,
        out_type=jax.ShapeDtypeStruct((NC, N, D), jnp.float32),
        mesh=mesh,
        scratch_types=[
            pltpu.VMEM((cpw, K), jnp.int32),
            pltpu.VMEM((ring, K), jnp.int32),
            pltpu.VMEM((ring * K,), jnp.float32),
            pltpu.VMEM((NB, K, D), jnp.float32),
            pltpu.VMEM_SHARED((N, D), jnp.float32),
            [pltpu.SemaphoreType.DMA] * NB,
            pltpu.SemaphoreType.DMA,
        ],
    )
    def k(h_hbm, col_hbm, row_hbm, val_hbm, out_hbm,
          col_v, dst_v, val_v, rows_v, acc, gsems, isem):
        cid = lax.axis_index("c")
        sid = lax.axis_index("s")
        wid = sid * NC + cid

        def zero_body(i, carry):
            for d in range(D // L):
                rows_v[0, i, pl.ds(d * L, L)] = jnp.zeros((L,), jnp.float32)
            return carry

        lax.fori_loop(0, K, zero_body, 0)
        base = jnp.minimum(sid * rps, N - rps)
        off = 0
        while off < rps:
            sz = min(K, rps - off)
            pltpu.sync_copy(rows_v.at[0, pl.ds(0, sz)],
                            acc.at[pl.ds(base + off, sz)])
            off += sz
        plsc.subcore_barrier()

        def idx_loads(g1):
            half = lax.rem(g1, 2) * IG
            return (
                (row_hbm.at[wid, pl.ds(g1 * IG, IG)],
                 dst_v.at[pl.ds(half, IG)]),
                (val_hbm.at[wid, pl.ds(g1 * IG * K, IG * K)],
                 val_v.at[pl.ds(half * K, IG * K)]),
            )

        def gather(ch, bb):
            return (h_hbm.at[col_v.at[ch]], rows_v.at[bb], gsems[bb])

        pltpu.sync_copy(col_hbm.at[wid], col_v)
        for src, dst in idx_loads(0):
            pltpu.sync_copy(src, dst)
        for src, dst in idx_loads(1):
            pltpu.async_copy(src, dst, isem)
        pltpu.async_copy(*gather(0, 0))
        pltpu.async_copy(*gather(1, 1))

        def pair_body(q, carry):
            for b in range(NB):
                ch = q * NB + b

                if b == 0:
                    gboundary = lax.rem(q, IG // NB) == 0

                    @pl.when(gboundary & (q > 0))
                    def _():
                        for src, dst in idx_loads(ch // IG):
                            pltpu.make_async_copy(src, dst, isem).wait()

                    @pl.when(gboundary & (ch + IG < cpw))
                    def _():
                        for src, dst in idx_loads(ch // IG + 1):
                            pltpu.async_copy(src, dst, isem)

                pltpu.make_async_copy(*gather(ch, b)).wait()

                rr = lax.rem(ch, ring)

                def scale_body(e16, c2, b=b, rr=rr):
                    vblock = val_v[pl.ds(rr * K + e16 * L, L)]
                    for j in range(L):
                        vv = jnp.full((L,), vblock[j])
                        e = e16 * L + j
                        for d in range(D // L):
                            sl = pl.ds(d * L, L)
                            rows_v[b, e, sl] = rows_v[b, e, sl] * vv
                    return c2

                lax.fori_loop(0, K // L, scale_body, 0)
                pltpu.sync_copy(rows_v.at[b], acc.at[dst_v.at[rr]], add=True)
                pltpu.async_copy(*gather(jnp.minimum(ch + 2, cpw - 1), b))
            return carry

        lax.fori_loop(0, cpw // NB, pair_body, 0)

        for b in range(NB):
            pltpu.make_async_copy(*gather(cpw - 1, b)).wait()
        plsc.subcore_barrier()
        pltpu.sync_copy(acc.at[pl.ds(base, rps)],
                        out_hbm.at[cid, pl.ds(base, rps)])

    return k(h, colw, roww, valw)


def _add_partials(p):
    _, N, D = p.shape

    def body(a_ref, b_ref, o_ref):
        o_ref[...] = a_ref[...] + b_ref[...]

    bn = N
    for cand in (2000, 1000, 500, 250, 128, 8):
        if N % cand == 0:
            bn = cand
            break
    grid = N // bn
    spec = pl.BlockSpec((bn, D), lambda i: (i, 0))
    return pl.pallas_call(
        body,
        out_shape=jax.ShapeDtypeStruct((N, D), jnp.float32),
        grid=(grid,),
        in_specs=[spec, spec],
        out_specs=spec,
    )(p[0], p[1])


def kernel(x, edge_index, edge_vals):
    N, D = x.shape
    E = edge_vals.shape[0]
    row = edge_index[0].astype(jnp.int32)
    col = edge_index[1].astype(jnp.int32)
    vals = edge_vals.astype(jnp.float32)

    cpw = -(-E // (NW * K))
    cpw = ((cpw + IG - 1) // IG) * IG
    epad = NW * K * cpw
    pad = epad - E
    if pad:
        row = jnp.concatenate([row, jnp.zeros((pad,), jnp.int32)])
        col = jnp.concatenate([col, jnp.zeros((pad,), jnp.int32)])
        vals = jnp.concatenate([vals, jnp.zeros((pad,), jnp.float32)])
    roww = row.reshape(NW, cpw, K)
    colw = col.reshape(NW, cpw, K)
    valw = vals.reshape(NW, cpw * K)

    out = x
    for _ in range(2):
        out = _add_partials(_sc_spmm(out, colw, roww, valw))
    return out


# speedup vs baseline: 1.3572x; 1.3572x over previous
import functools

import jax
import jax.numpy as jnp
from jax import lax
from jax.experimental import pallas as pl
from jax.experimental.pallas import tpu as pltpu
from jax.experimental.pallas import tpu_sc as plsc

NC = 2
NS = 16
L = 16
NW = NC * NS
K = 128
NSTR = 2


def _sc_spmm(h, colw, roww, valw):
    N, D = h.shape
    cpw = colw.shape[1]
    ks = K // NSTR
    mesh = plsc.VectorSubcoreMesh(core_axis_name="c", subcore_axis_name="s")
    rps = ((-(-N // NS)) + 7) // 8 * 8

    @functools.partial(
        pl.kernel,
        out_type=jax.ShapeDtypeStruct((NC, N, D), jnp.float32),
        mesh=mesh,
        scratch_types=[
            pltpu.VMEM((cpw, K), jnp.int32),
            pltpu.VMEM((cpw, K), jnp.int32),
            pltpu.VMEM((cpw * K,), jnp.float32),
            pltpu.VMEM((K, D), jnp.float32),
            pltpu.VMEM_SHARED((N, D), jnp.float32),
            [pltpu.SemaphoreType.DMA] * NSTR,
        ],
    )
    def k(h_hbm, col_hbm, row_hbm, val_hbm, out_hbm,
          col_v, dst_v, val_v, rows_v, acc, sems):
        cid = lax.axis_index("c")
        sid = lax.axis_index("s")
        wid = sid * NC + cid

        def zero_body(i, carry):
            for d in range(D // L):
                rows_v[i, pl.ds(d * L, L)] = jnp.zeros((L,), jnp.float32)
            return carry

        lax.fori_loop(0, K, zero_body, 0)
        base = jnp.minimum(sid * rps, N - rps)
        off = 0
        while off < rps:
            sz = min(K, rps - off)
            pltpu.sync_copy(rows_v.at[pl.ds(0, sz)],
                            acc.at[pl.ds(base + off, sz)])
            off += sz
        plsc.subcore_barrier()

        pltpu.sync_copy(col_hbm.at[wid], col_v)
        pltpu.sync_copy(row_hbm.at[wid], dst_v)
        pltpu.sync_copy(val_hbm.at[wid], val_v)

        def chunk_body(g, carry):
            descs = [
                pltpu.async_copy(
                    h_hbm.at[col_v.at[g, pl.ds(s * ks, ks)]],
                    rows_v.at[pl.ds(s * ks, ks)], sems[s])
                for s in range(NSTR)
            ]
            for d in descs:
                d.wait()

            def scale_body(e16, c2):
                vblock = val_v[pl.ds(g * K + e16 * L, L)]
                for j in range(L):
                    vv = jnp.full((L,), vblock[j])
                    e = e16 * L + j
                    for d in range(D // L):
                        sl = pl.ds(d * L, L)
                        rows_v[e, sl] = rows_v[e, sl] * vv
                return c2

            lax.fori_loop(0, K // L, scale_body, 0)

            pltpu.sync_copy(rows_v, acc.at[dst_v.at[g]], add=True)
            return carry

        lax.fori_loop(0, cpw, chunk_body, 0)
        plsc.subcore_barrier()

        pltpu.sync_copy(acc.at[pl.ds(base, rps)],
                        out_hbm.at[cid, pl.ds(base, rps)])

    return k(h, colw, roww, valw)


def _add_partials(p):
    _, N, D = p.shape

    def body(a_ref, b_ref, o_ref):
        o_ref[...] = a_ref[...] + b_ref[...]

    bn = N
    for cand in (2000, 1000, 500, 250, 128, 8):
        if N % cand == 0:
            bn = cand
            break
    grid = N // bn
    spec = pl.BlockSpec((bn, D), lambda i: (i, 0))
    return pl.pallas_call(
        body,
        out_shape=jax.ShapeDtypeStruct((N, D), jnp.float32),
        grid=(grid,),
        in_specs=[spec, spec],
        out_specs=spec,
    )(p[0], p[1])


def kernel(x, edge_index, edge_vals):
    N, D = x.shape
    E = edge_vals.shape[0]
    row = edge_index[0].astype(jnp.int32)
    col = edge_index[1].astype(jnp.int32)
    vals = edge_vals.astype(jnp.float32)

    cpw = -(-E // (NW * K))
    epad = NW * K * cpw
    pad = epad - E
    if pad:
        row = jnp.concatenate([row, jnp.zeros((pad,), jnp.int32)])
        col = jnp.concatenate([col, jnp.zeros((pad,), jnp.int32)])
        vals = jnp.concatenate([vals, jnp.zeros((pad,), jnp.float32)])
    roww = row.reshape(NW, cpw, K)
    colw = col.reshape(NW, cpw, K)
    valw = vals.reshape(NW, cpw * K)

    out = x
    for _ in range(2):
        out = _add_partials(_sc_spmm(out, colw, roww, valw))
    return out


# speedup vs baseline: 1.3588x; 1.0011x over previous
import functools

import jax
import jax.numpy as jnp
from jax import lax
from jax.experimental import pallas as pl
from jax.experimental.pallas import tpu as pltpu
from jax.experimental.pallas import tpu_sc as plsc

NC = 2
NS = 16
L = 16
NW = NC * NS
K = 128
NSTR = 2


def _sc_spmm(h, colw, roww, valw):
    N, D = h.shape
    cpw = colw.shape[1]
    ks = K // NSTR
    mesh = plsc.VectorSubcoreMesh(core_axis_name="c", subcore_axis_name="s")
    rps = ((-(-N // NS)) + 7) // 8 * 8

    @functools.partial(
        pl.kernel,
        out_type=jax.ShapeDtypeStruct((NC, N, D), jnp.float32),
        mesh=mesh,
        scratch_types=[
            pltpu.VMEM((cpw, K), jnp.int32),
            pltpu.VMEM((cpw, K), jnp.int32),
            pltpu.VMEM((cpw * K,), jnp.float32),
            pltpu.VMEM((K, D), jnp.float32),
            pltpu.VMEM_SHARED((N, D), jnp.float32),
            [pltpu.SemaphoreType.DMA] * NSTR,
        ],
    )
    def k(h_hbm, col_hbm, row_hbm, val_hbm, out_hbm,
          col_v, dst_v, val_v, rows_v, acc, sems):
        cid = lax.axis_index("c")
        sid = lax.axis_index("s")
        wid = sid * NC + cid

        def zero_body(i, carry):
            for d in range(D // L):
                rows_v[i, pl.ds(d * L, L)] = jnp.zeros((L,), jnp.float32)
            return carry

        lax.fori_loop(0, K, zero_body, 0)
        base = jnp.minimum(sid * rps, N - rps)
        off = 0
        while off < rps:
            sz = min(K, rps - off)
            pltpu.sync_copy(rows_v.at[pl.ds(0, sz)],
                            acc.at[pl.ds(base + off, sz)])
            off += sz
        plsc.subcore_barrier()

        pltpu.sync_copy(col_hbm.at[wid], col_v)
        pltpu.sync_copy(row_hbm.at[wid], dst_v)
        pltpu.sync_copy(val_hbm.at[wid], val_v)

        def chunk_body(g, carry):
            descs = [
                pltpu.async_copy(
                    h_hbm.at[col_v.at[g, pl.ds(s * ks, ks)]],
                    rows_v.at[pl.ds(s * ks, ks)], sems[s])
                for s in range(NSTR)
            ]
            for d in descs:
                d.wait()

            def scale_body(e16, c2):
                va = val_v[pl.ds(g * K + e16 * L, L)]
                for j in range(L):
                    vv = jnp.full((L,), va[j])
                    e = e16 * L + j
                    for d in range(D // L):
                        rows_v[e, pl.ds(d * L, L)] = (
                            rows_v[e, pl.ds(d * L, L)] * vv)
                return c2

            lax.fori_loop(0, K // L, scale_body, 0)

            pltpu.sync_copy(rows_v, acc.at[dst_v.at[g]], add=True)
            return carry

        lax.fori_loop(0, cpw, chunk_body, 0)
        plsc.subcore_barrier()

        pltpu.sync_copy(acc.at[pl.ds(base, rps)],
                        out_hbm.at[cid, pl.ds(base, rps)])

    return k(h, colw, roww, valw)


def _add_partials(p):
    _, N, D = p.shape

    def body(a_ref, b_ref, o_ref):
        o_ref[...] = a_ref[...] + b_ref[...]

    bn = N
    for cand in (2000, 1000, 500, 250, 128, 8):
        if N % cand == 0:
            bn = cand
            break
    grid = N // bn
    spec = pl.BlockSpec((bn, D), lambda i: (i, 0))
    return pl.pallas_call(
        body,
        out_shape=jax.ShapeDtypeStruct((N, D), jnp.float32),
        grid=(grid,),
        in_specs=[spec, spec],
        out_specs=spec,
    )(p[0], p[1])


def kernel(x, edge_index, edge_vals):
    N, D = x.shape
    E = edge_vals.shape[0]
    row = edge_index[0].astype(jnp.int32)
    col = edge_index[1].astype(jnp.int32)
    vals = edge_vals.astype(jnp.float32)

    cpw = -(-E // (NW * K))
    epad = NW * K * cpw
    pad = epad - E
    if pad:
        row = jnp.concatenate([row, jnp.zeros((pad,), jnp.int32)])
        col = jnp.concatenate([col, jnp.zeros((pad,), jnp.int32)])
        vals = jnp.concatenate([vals, jnp.zeros((pad,), jnp.float32)])
    roww = row.reshape(NW, cpw, K)
    colw = col.reshape(NW, cpw, K)
    valw = vals.reshape(NW, cpw * K)

    out = x.astype(jnp.float32)
    for _ in range(2):
        out = _add_partials(_sc_spmm(out, colw, roww, valw))
    return out
